# initial kernel scaffold (unmeasured)
import jax
import jax.numpy as jnp
from jax import lax
from jax.experimental import pallas as pl
from jax.experimental.pallas import tpu as pltpu

N_DEV = 4
B, Sq, Skv, Hq, Dh = 2, 512, 512, 32, 64
H_LOC = Hq // N_DEV
D_LOC = H_LOC * Dh


def kernel(x, Wq, K_ext, V_ext, Wo):
    K2 = K_ext.reshape(B, Skv, Hq * Dh)
    V2 = V_ext.reshape(B, Skv, Hq * Dh)

    def body(x_ref, wq_ref, k_ref, v_ref, wo_ref, out_ref,
             comm_ref, ctx_ref, send_sems, recv_sems):
        my_i = lax.axis_index("i")
        left = lax.rem(my_i - 1 + N_DEV, N_DEV)
        right = lax.rem(my_i + 1, N_DEV)

        barrier_sem = pltpu.get_barrier_semaphore()
        for nbr in [left, right]:
            pl.semaphore_signal(
                barrier_sem, inc=1,
                device_id=(nbr,), device_id_type=pl.DeviceIdType.MESH,
            )
        pl.semaphore_wait(barrier_sem, 2)

        qi = lax.broadcasted_iota(jnp.int32, (Sq, Skv), 0)
        ki = lax.broadcasted_iota(jnp.int32, (Sq, Skv), 1)
        mask = (jnp.abs(qi - ki) <= 128) | (ki < 32) | (qi < 32)
        neg_bias = jnp.where(mask, 0.0, -1e9).astype(jnp.float32)

        head_off = my_i * D_LOC

        for b in range(B):
            xb = x_ref[b].astype(jnp.bfloat16)
            wq = wq_ref[:, :].astype(jnp.bfloat16)
            qb = jnp.dot(xb, wq, preferred_element_type=jnp.float32)
            qb = qb.astype(jnp.bfloat16)

            kb = pl.load(k_ref, (b, slice(None), pl.ds(head_off, D_LOC)))
            vb = pl.load(v_ref, (b, slice(None), pl.ds(head_off, D_LOC)))
            kb = kb.astype(jnp.bfloat16)
            vb = vb.astype(jnp.bfloat16)

            for h in range(H_LOC):
                sl = slice(h * Dh, (h + 1) * Dh)
                q_h = qb[:, sl]
                k_h = kb[:, sl]
                v_h = vb[:, sl]
                scores = jnp.dot(q_h, k_h.T,
                                 preferred_element_type=jnp.float32)
                scores = scores * 0.125 + neg_bias
                m = jnp.max(scores, axis=-1, keepdims=True)
                w = jnp.exp(scores - m)
                w = w / jnp.sum(w, axis=-1, keepdims=True)
                ctx_h = jnp.dot(w.astype(jnp.bfloat16), v_h,
                                preferred_element_type=jnp.float32)
                ctx_ref[:, sl] = ctx_h.astype(jnp.bfloat16)

            wo = wo_ref[:, :].astype(jnp.bfloat16)
            partial_b = jnp.dot(ctx_ref[:, :], wo,
                                preferred_element_type=jnp.float32)
            out_ref[b] = partial_b
            comm_ref[0, b] = partial_b.astype(jnp.bfloat16)

        for h in range(N_DEV - 1):
            rdma = pltpu.make_async_remote_copy(
                src_ref=comm_ref.at[h],
                dst_ref=comm_ref.at[h + 1],
                send_sem=send_sems.at[h],
                recv_sem=recv_sems.at[h],
                device_id=(right,),
                device_id_type=pl.DeviceIdType.MESH,
            )
            rdma.start()
            rdma.wait()
            for b in range(B):
                out_ref[b] += comm_ref[h + 1, b].astype(jnp.float32)

    return pl.pallas_call(
        body,
        out_shape=jax.ShapeDtypeStruct((B, Sq, 768), jnp.float32),
        in_specs=[pl.BlockSpec(memory_space=pltpu.VMEM)] * 5,
        out_specs=pl.BlockSpec(memory_space=pltpu.VMEM),
        scratch_shapes=[
            pltpu.VMEM((N_DEV, B, Sq, 768), jnp.bfloat16),
            pltpu.VMEM((Sq, D_LOC), jnp.bfloat16),
            pltpu.SemaphoreType.DMA((N_DEV - 1,)),
            pltpu.SemaphoreType.DMA((N_DEV - 1,)),
        ],
        compiler_params=pltpu.CompilerParams(collective_id=0),
    )(x, Wq, K2, V2, Wo)


# baseline (device time: 92567 ns/iter reference)
import jax
import jax.numpy as jnp
from jax import lax
from jax.experimental import pallas as pl
from jax.experimental.pallas import tpu as pltpu

N_DEV = 4
B, Sq, Skv, Hq, Dh = 2, 512, 512, 32, 64
H_LOC = Hq // N_DEV
D_LOC = H_LOC * Dh


def kernel(x, Wq, K_ext, V_ext, Wo):
    K2 = K_ext.reshape(B, Skv, Hq * Dh)
    V2 = V_ext.reshape(B, Skv, Hq * Dh)

    def body(x_ref, wq_ref, k_ref, v_ref, wo_ref, out_ref,
             comm_ref, ctx_ref, send_sems, recv_sems):
        my_i = lax.axis_index("i")
        left = lax.rem(my_i - 1 + N_DEV, N_DEV)
        right = lax.rem(my_i + 1, N_DEV)

        barrier_sem = pltpu.get_barrier_semaphore()
        for nbr in [left, right]:
            pl.semaphore_signal(
                barrier_sem, inc=1,
                device_id=(nbr,), device_id_type=pl.DeviceIdType.MESH,
            )
        pl.semaphore_wait(barrier_sem, 2)

        qi = lax.broadcasted_iota(jnp.int32, (Sq, Skv), 0)
        ki = lax.broadcasted_iota(jnp.int32, (Sq, Skv), 1)
        mask = (jnp.abs(qi - ki) <= 128) | (ki < 32) | (qi < 32)
        neg_bias = jnp.where(mask, 0.0, -1e9).astype(jnp.float32)

        head_off = my_i * D_LOC

        for b in range(B):
            xb = x_ref[b].astype(jnp.bfloat16)
            wq = wq_ref[:, :].astype(jnp.bfloat16)
            qb = jnp.dot(xb, wq, preferred_element_type=jnp.float32)
            qb = qb.astype(jnp.bfloat16)

            kb = k_ref[b, :, pl.ds(head_off, D_LOC)].astype(jnp.bfloat16)
            vb = v_ref[b, :, pl.ds(head_off, D_LOC)].astype(jnp.bfloat16)

            for h in range(H_LOC):
                sl = slice(h * Dh, (h + 1) * Dh)
                q_h = qb[:, sl]
                k_h = kb[:, sl]
                v_h = vb[:, sl]
                scores = jnp.dot(q_h, k_h.T,
                                 preferred_element_type=jnp.float32)
                scores = scores * 0.125 + neg_bias
                m = jnp.max(scores, axis=-1, keepdims=True)
                w = jnp.exp(scores - m)
                w = w / jnp.sum(w, axis=-1, keepdims=True)
                ctx_h = jnp.dot(w.astype(jnp.bfloat16), v_h,
                                preferred_element_type=jnp.float32)
                ctx_ref[:, sl] = ctx_h.astype(jnp.bfloat16)

            wo = wo_ref[:, :].astype(jnp.bfloat16)
            partial_b = jnp.dot(ctx_ref[:, :], wo,
                                preferred_element_type=jnp.float32)
            out_ref[b] = partial_b
            comm_ref[0, b] = partial_b.astype(jnp.bfloat16)

        for h in range(N_DEV - 1):
            rdma = pltpu.make_async_remote_copy(
                src_ref=comm_ref.at[h],
                dst_ref=comm_ref.at[h + 1],
                send_sem=send_sems.at[h],
                recv_sem=recv_sems.at[h],
                device_id=(right,),
                device_id_type=pl.DeviceIdType.MESH,
            )
            rdma.start()
            rdma.wait()
            for b in range(B):
                out_ref[b] += comm_ref[h + 1, b].astype(jnp.float32)

    return pl.pallas_call(
        body,
        out_shape=jax.ShapeDtypeStruct((B, Sq, 768), jnp.float32),
        in_specs=[pl.BlockSpec(memory_space=pltpu.VMEM)] * 5,
        out_specs=pl.BlockSpec(memory_space=pltpu.VMEM),
        scratch_shapes=[
            pltpu.VMEM((N_DEV, B, Sq, 768), jnp.bfloat16),
            pltpu.VMEM((Sq, D_LOC), jnp.bfloat16),
            pltpu.SemaphoreType.DMA((N_DEV - 1,)),
            pltpu.SemaphoreType.DMA((N_DEV - 1,)),
        ],
        compiler_params=pltpu.CompilerParams(collective_id=0),
    )(x, Wq, K2, V2, Wo)
